# baseline (speedup 1.0000x reference)
import functools

import jax
import jax.numpy as jnp
from jax import lax
from jax.experimental import pallas as pl
from jax.experimental.pallas import tpu as pltpu
from jax.experimental.pallas import tpu_sc as plsc

B, S, D, V = 4096, 200, 64, 1000000
N = B * S
NC, NS = 2, 16
NW = NC * NS
PER_W = N // NW
CHUNK = 800
NCH = PER_W // CHUNK
GS = 100
NG = CHUNK // GS
IDX_COLS = 100

_mesh = plsc.VectorSubcoreMesh(core_axis_name="c", subcore_axis_name="s")


@functools.partial(
    pl.kernel,
    out_type=jax.ShapeDtypeStruct((N, D), jnp.float32),
    mesh=_mesh,
    scratch_types=[
        pltpu.VMEM((NG, GS), jnp.int32),
        pltpu.VMEM((CHUNK, D), jnp.float32),
        pltpu.VMEM((S, D), jnp.float32),
        pltpu.SemaphoreType.DMA,
    ],
    compiler_params=pltpu.CompilerParams(use_tc_tiling_on_sc=False),
)
def _emb(idx_hbm, table_hbm, pos_hbm, out_hbm, idx_v, rows_v, pos_v, sem):
    wid = lax.axis_index("s") * NC + lax.axis_index("c")
    pltpu.sync_copy(pos_hbm, pos_v)

    def chunk_body(ch, _):
        base = pl.multiple_of(wid * PER_W + ch * CHUNK, CHUNK)
        idx_row = pl.multiple_of(base // IDX_COLS, NG)
        pltpu.sync_copy(idx_hbm.at[pl.ds(idx_row, NG)], idx_v)
        copies = [
            pltpu.async_copy(
                table_hbm.at[idx_v.at[j]],
                rows_v.at[pl.ds(j * GS, GS)],
                sem,
            )
            for j in range(NG)
        ]
        for c in copies:
            c.wait()

        def add_body(r, _):
            for half in range(CHUNK // S):
                for c in range(D // 16):
                    sl = pl.ds(c * 16, 16)
                    rows_v[half * S + r, sl] = rows_v[half * S + r, sl] + pos_v[r, sl]
            return 0

        lax.fori_loop(0, S, add_body, 0)
        pltpu.sync_copy(rows_v, out_hbm.at[pl.ds(base, CHUNK)])
        return 0

    lax.fori_loop(0, NCH, chunk_body, 0)


def kernel(inputs, token_table, pos_table):
    idx = inputs.astype(jnp.int32).reshape(N // IDX_COLS, IDX_COLS)
    out = _emb(idx, token_table, pos_table)
    return out.reshape(B, S, D)


# speedup vs baseline: 1.0898x; 1.0898x over previous
import functools

import jax
import jax.numpy as jnp
from jax import lax
from jax.experimental import pallas as pl
from jax.experimental.pallas import tpu as pltpu
from jax.experimental.pallas import tpu_sc as plsc

B, S, D, V = 4096, 200, 64, 1000000
N = B * S
NC, NS = 2, 16
NW = NC * NS
PER_W = N // NW
CHUNK = 800
NCH = PER_W // CHUNK
GS = 100
NG = CHUNK // GS
IDX_COLS = 100
HALVES = CHUNK // S

_mesh = plsc.VectorSubcoreMesh(core_axis_name="c", subcore_axis_name="s")


@functools.partial(
    pl.kernel,
    out_type=jax.ShapeDtypeStruct((N, D), jnp.float32),
    mesh=_mesh,
    scratch_types=[
        pltpu.VMEM((2, NG, GS), jnp.int32),
        pltpu.VMEM((2, CHUNK, D), jnp.float32),
        pltpu.VMEM((S, D), jnp.float32),
        pltpu.SemaphoreType.DMA,
        pltpu.SemaphoreType.DMA,
    ],
    compiler_params=pltpu.CompilerParams(use_tc_tiling_on_sc=False),
)
def _emb(idx_hbm, table_hbm, pos_hbm, out_hbm, idx_v, rows_v, pos_v, sem0, sem1):
    wid = lax.axis_index("s") * NC + lax.axis_index("c")
    sems = (sem0, sem1)

    def chunk_base(ch):
        return pl.multiple_of(wid * PER_W + ch * CHUNK, CHUNK)

    def fire(buf, ch):
        base = chunk_base(ch)
        idx_row = pl.multiple_of(base // IDX_COLS, NG)
        pltpu.sync_copy(idx_hbm.at[pl.ds(idx_row, NG)], idx_v.at[buf])
        for j in range(NG):
            pltpu.async_copy(
                table_hbm.at[idx_v.at[buf, j]],
                rows_v.at[buf, pl.ds(j * GS, GS)],
                sems[buf],
            )

    def drain(buf):
        for j in range(NG):
            pltpu.make_async_copy(
                table_hbm.at[idx_v.at[buf, j]],
                rows_v.at[buf, pl.ds(j * GS, GS)],
                sems[buf],
            ).wait()

    pltpu.sync_copy(pos_hbm, pos_v)
    fire(0, 0)
    fire(1, 1)

    def body2(i, _):
        for buf in (0, 1):
            ch = 2 * i + buf
            drain(buf)

            @plsc.parallel_loop(0, S, unroll=2)
            def _add(r):
                for c in range(D // 16):
                    sl = pl.ds(c * 16, 16)
                    pv = pos_v[r, sl]
                    for h in range(HALVES):
                        rows_v[buf, h * S + r, sl] = rows_v[buf, h * S + r, sl] + pv

            pltpu.sync_copy(rows_v.at[buf], out_hbm.at[pl.ds(chunk_base(ch), CHUNK)])

            @pl.when(ch + 2 < NCH)
            def _():
                fire(buf, ch + 2)

        return 0

    lax.fori_loop(0, NCH // 2, body2, 0)


def kernel(inputs, token_table, pos_table):
    idx = inputs.astype(jnp.int32).reshape(N // IDX_COLS, IDX_COLS)
    out = _emb(idx, token_table, pos_table)
    return out.reshape(B, S, D)
